# Initial kernel scaffold; baseline (speedup 1.0000x reference)
import functools

import jax
import jax.numpy as jnp
from jax.experimental import pallas as pl
from jax.experimental.pallas import tpu as pltpu

_D_IN = 128
_D_OUT = 128
_NK = 64
_PHI_MIN = -0.1
_PHI_MAX = 1.1 + 0.02 * (_D_OUT - 1)
_PHI2_MIN = -5.0
_PHI2_MAX = 5.0
_PHI_DELTA = (_PHI_MAX - _PHI_MIN) / (_NK - 1)
_PHI2_DELTA = (_PHI2_MAX - _PHI2_MIN) / (_NK - 1)

_B_BLK = 8


def _coeff_tables(values, delta):
    v = values.astype(jnp.float32)
    i = jnp.arange(_NK)
    vm1 = v[jnp.clip(i - 1, 0, _NK - 1)]
    vp1 = v[jnp.clip(i + 1, 0, _NK - 1)]
    vp2 = v[jnp.clip(i + 2, 0, _NK - 1)]
    b = 0.5 * (vp1 - vm1)
    mh1 = 0.5 * (vp2 - v)
    c = -3.0 * v - 2.0 * b + 3.0 * vp1 - mh1
    d = 2.0 * v + b - 2.0 * vp1 + mh1
    return v, b, c, d


def _lane_gather(tab_row, idx):
    tab = jnp.broadcast_to(tab_row, (idx.shape[0], _NK))
    return jnp.take_along_axis(tab, idx, axis=-1)


def _block_kernel(params_ref, tabs_ref, x_ref, lam_ref, xo_ref, out_ref):
    eta = params_ref[0, 0]
    rw = params_ref[0, 1]
    ml2 = params_ref[0, 2]
    mr2 = params_ref[0, 3]

    x2 = x_ref[...]

    inv_d1 = jnp.float32(1.0 / _PHI_DELTA)
    q = jax.lax.broadcasted_iota(jnp.float32, (1, _D_OUT), 1)
    uq = (eta * inv_d1) * q
    xu = (x2 - jnp.float32(_PHI_MIN)) * inv_d1
    u = xu + uq
    idx_f = jnp.floor(u)
    idx_f = jnp.clip(idx_f, 0.0, float(_NK - 2))
    idx = idx_f.astype(jnp.int32)
    t = u - idx_f

    a = _lane_gather(tabs_ref[0:1, :], idx)
    b = _lane_gather(tabs_ref[1:2, :], idx)
    c = _lane_gather(tabs_ref[2:3, :], idx)
    d = _lane_gather(tabs_ref[3:4, :], idx)
    phi = a + t * (b + t * (c + t * d))

    lam = lam_ref[...].reshape(1, _D_IN, 1)
    phi3 = phi.reshape(_B_BLK, _D_IN, _D_OUT)
    s = jnp.sum(phi3 * lam, axis=1)

    inv_d2 = jnp.float32(1.0 / _PHI2_DELTA)
    sc = jnp.clip(s, jnp.float32(_PHI2_MIN), jnp.float32(_PHI2_MAX))
    u2 = (sc - jnp.float32(_PHI2_MIN)) * inv_d2
    idx2_f = jnp.clip(jnp.floor(u2), 0.0, float(_NK - 2))
    idx2 = idx2_f.astype(jnp.int32)
    t2 = u2 - idx2_f
    a2 = _lane_gather(tabs_ref[4:5, :], idx2)
    b2 = _lane_gather(tabs_ref[5:6, :], idx2)
    c2 = _lane_gather(tabs_ref[6:7, :], idx2)
    d2 = _lane_gather(tabs_ref[7:8, :], idx2)
    y = a2 + t2 * (b2 + t2 * (c2 + t2 * d2))
    zero = jnp.float32(0.0)
    y = y + jnp.where(s < jnp.float32(_PHI2_MIN), ml2 * (s - jnp.float32(_PHI2_MIN)), zero)
    y = y + jnp.where(s > jnp.float32(_PHI2_MAX), mr2 * (s - jnp.float32(_PHI2_MAX)), zero)

    out_ref[...] = y + rw * xo_ref[...]


@jax.jit
def kernel(x, x_original, phi_values, Phi_values, lambdas, eta, residual_weight):
    batch = x.shape[0]
    n_blk = batch // _B_BLK

    pa, pb, pc, pd = _coeff_tables(phi_values, _PHI_DELTA)
    qa, qb, qc, qd = _coeff_tables(Phi_values, _PHI2_DELTA)
    tabs = jnp.stack([pa, pb, pc, pd, qa, qb, qc, qd])

    Pv = Phi_values.astype(jnp.float32)
    ml2 = (Pv[1] - Pv[0]) / jnp.float32(_PHI2_DELTA)
    mr2 = (Pv[-1] - Pv[-2]) / jnp.float32(_PHI2_DELTA)
    params = jnp.stack([eta.astype(jnp.float32),
                        residual_weight.astype(jnp.float32),
                        ml2, mr2]).reshape(1, 4)

    x_col = x.reshape(batch * _D_IN, 1)
    lam_col = lambdas.astype(jnp.float32).reshape(_D_IN, 1)

    r = _B_BLK * _D_IN
    out = pl.pallas_call(
        _block_kernel,
        grid=(n_blk,),
        in_specs=[
            pl.BlockSpec(memory_space=pltpu.SMEM),
            pl.BlockSpec((8, _NK), lambda i: (0, 0)),
            pl.BlockSpec((r, 1), lambda i: (i, 0)),
            pl.BlockSpec((_D_IN, 1), lambda i: (0, 0)),
            pl.BlockSpec((_B_BLK, _D_OUT), lambda i: (i, 0)),
        ],
        out_specs=pl.BlockSpec((_B_BLK, _D_OUT), lambda i: (i, 0)),
        out_shape=jax.ShapeDtypeStruct((batch, _D_OUT), jnp.float32),
        compiler_params=pltpu.CompilerParams(
            dimension_semantics=("arbitrary",),
        ),
    )(params, tabs, x_col, lam_col, x_original)
    return out


# speedup vs baseline: 39.5868x; 39.5868x over previous
import functools

import jax
import jax.numpy as jnp
from jax.experimental import pallas as pl
from jax.experimental.pallas import tpu as pltpu

_D_IN = 128
_D_OUT = 128
_NK = 64
_PHI_MIN = -0.1
_PHI_MAX = 1.1 + 0.02 * (_D_OUT - 1)
_PHI2_MIN = -5.0
_PHI2_MAX = 5.0
_PHI_DELTA = (_PHI_MAX - _PHI_MIN) / (_NK - 1)
_PHI2_DELTA = (_PHI2_MAX - _PHI2_MIN) / (_NK - 1)

_B_BLK = 8


def _coeff_tables(values, delta):
    v = values.astype(jnp.float32)
    i = jnp.arange(_NK)
    vm1 = v[jnp.clip(i - 1, 0, _NK - 1)]
    vp1 = v[jnp.clip(i + 1, 0, _NK - 1)]
    vp2 = v[jnp.clip(i + 2, 0, _NK - 1)]
    b = 0.5 * (vp1 - vm1)
    mh1 = 0.5 * (vp2 - v)
    c = -3.0 * v - 2.0 * b + 3.0 * vp1 - mh1
    d = 2.0 * v + b - 2.0 * vp1 + mh1
    return v, b, c, d


def _lane_gather(tab_row, idx):
    tab = jnp.broadcast_to(tab_row, (idx.shape[0], _NK))
    return jnp.take_along_axis(tab, idx, axis=-1)


def _block_kernel(params_ref, tabs_ref, x_ref, lam_ref, xo_ref, out_ref):
    eta = params_ref[0, 0]
    rw = params_ref[0, 1]
    ml2 = params_ref[0, 2]
    mr2 = params_ref[0, 3]

    x2 = x_ref[...]

    inv_d1 = jnp.float32(1.0 / _PHI_DELTA)
    q = jax.lax.broadcasted_iota(jnp.int32, (1, _D_OUT), 1).astype(jnp.float32)
    uq = (eta * inv_d1) * q
    xu = (x2 - jnp.float32(_PHI_MIN)) * inv_d1
    u = xu + uq
    idx_f = jnp.floor(u)
    idx_f = jnp.clip(idx_f, 0.0, float(_NK - 2))
    idx = idx_f.astype(jnp.int32)
    t = u - idx_f

    a = _lane_gather(tabs_ref[0:1, :], idx)
    b = _lane_gather(tabs_ref[1:2, :], idx)
    c = _lane_gather(tabs_ref[2:3, :], idx)
    d = _lane_gather(tabs_ref[3:4, :], idx)
    phi = a + t * (b + t * (c + t * d))

    lam = lam_ref[...].reshape(1, _D_IN, 1)
    phi3 = phi.reshape(_B_BLK, _D_IN, _D_OUT)
    s = jnp.sum(phi3 * lam, axis=1)

    inv_d2 = jnp.float32(1.0 / _PHI2_DELTA)
    sc = jnp.clip(s, jnp.float32(_PHI2_MIN), jnp.float32(_PHI2_MAX))
    u2 = (sc - jnp.float32(_PHI2_MIN)) * inv_d2
    idx2_f = jnp.clip(jnp.floor(u2), 0.0, float(_NK - 2))
    idx2 = idx2_f.astype(jnp.int32)
    t2 = u2 - idx2_f
    a2 = _lane_gather(tabs_ref[4:5, :], idx2)
    b2 = _lane_gather(tabs_ref[5:6, :], idx2)
    c2 = _lane_gather(tabs_ref[6:7, :], idx2)
    d2 = _lane_gather(tabs_ref[7:8, :], idx2)
    y = a2 + t2 * (b2 + t2 * (c2 + t2 * d2))
    zero = jnp.float32(0.0)
    y = y + jnp.where(s < jnp.float32(_PHI2_MIN), ml2 * (s - jnp.float32(_PHI2_MIN)), zero)
    y = y + jnp.where(s > jnp.float32(_PHI2_MAX), mr2 * (s - jnp.float32(_PHI2_MAX)), zero)

    out_ref[...] = y + rw * xo_ref[...]


@jax.jit
def kernel(x, x_original, phi_values, Phi_values, lambdas, eta, residual_weight):
    batch = x.shape[0]
    n_blk = batch // _B_BLK

    pa, pb, pc, pd = _coeff_tables(phi_values, _PHI_DELTA)
    qa, qb, qc, qd = _coeff_tables(Phi_values, _PHI2_DELTA)
    tabs = jnp.stack([pa, pb, pc, pd, qa, qb, qc, qd])

    Pv = Phi_values.astype(jnp.float32)
    ml2 = (Pv[1] - Pv[0]) / jnp.float32(_PHI2_DELTA)
    mr2 = (Pv[-1] - Pv[-2]) / jnp.float32(_PHI2_DELTA)
    params = jnp.stack([eta.astype(jnp.float32),
                        residual_weight.astype(jnp.float32),
                        ml2, mr2]).reshape(1, 4)

    x_col = x.reshape(batch * _D_IN, 1)
    lam_col = lambdas.astype(jnp.float32).reshape(_D_IN, 1)

    r = _B_BLK * _D_IN
    out = pl.pallas_call(
        _block_kernel,
        grid=(n_blk,),
        in_specs=[
            pl.BlockSpec(memory_space=pltpu.SMEM),
            pl.BlockSpec((8, _NK), lambda i: (0, 0)),
            pl.BlockSpec((r, 1), lambda i: (i, 0)),
            pl.BlockSpec((_D_IN, 1), lambda i: (0, 0)),
            pl.BlockSpec((_B_BLK, _D_OUT), lambda i: (i, 0)),
        ],
        out_specs=pl.BlockSpec((_B_BLK, _D_OUT), lambda i: (i, 0)),
        out_shape=jax.ShapeDtypeStruct((batch, _D_OUT), jnp.float32),
        compiler_params=pltpu.CompilerParams(
            dimension_semantics=("arbitrary",),
        ),
    )(params, tabs, x_col, lam_col, x_original)
    return out
